# Initial kernel scaffold; baseline (speedup 1.0000x reference)
import functools

import jax
import jax.numpy as jnp
from jax import lax
from jax.experimental import pallas as pl
from jax.experimental.pallas import tpu as pltpu
from jax.experimental.pallas import tpu_sc as plsc

N = 50000
E = 800000
H = 64
IN = 6

NC = 2
NS = 16

HALF = 25600
NP = NC * HALF
RPS = HALF // NS
ACC_ROWS = HALF + 16
ZR = ACC_ROWS // NS

C = 128
EPT = 51200
EP = NS * EPT

BR = 512

_mesh = plsc.VectorSubcoreMesh(
    core_axis_name="c", subcore_axis_name="s", num_cores=NC, num_subcores=NS
)


def _deg_body(dst_hbm, zeros_hbm, ones_hbm, deg_hbm, accd, dst_buf, idx_buf,
              ones_buf):
    c = lax.axis_index("c")
    s = lax.axis_index("s")
    base = c * HALF
    pltpu.sync_copy(zeros_hbm, accd.at[pl.ds(s * ZR, ZR)])
    pltpu.sync_copy(ones_hbm, ones_buf)
    plsc.subcore_barrier()

    def chunk(j, carry):
        off = s * EPT + j * C
        pltpu.sync_copy(dst_hbm.at[pl.ds(off, C)], dst_buf)
        for k in range(C // 16):
            d = dst_buf[pl.ds(k * 16, 16)] - base
            ok = (d >= 0) & (d < HALF)
            idx_buf[pl.ds(k * 16, 16)] = jnp.where(ok, d, HALF)
        pltpu.sync_copy(ones_buf, accd.at[idx_buf], add=True)
        return carry

    lax.fori_loop(0, EPT // C, chunk, 0)
    plsc.subcore_barrier()
    pltpu.sync_copy(accd.at[pl.ds(s * RPS, RPS)],
                    deg_hbm.at[pl.ds(base + s * RPS, RPS)])


_deg_kernel = functools.partial(
    pl.kernel,
    out_type=jax.ShapeDtypeStruct((NP, 16), jnp.float32),
    mesh=_mesh,
    scratch_types=[
        pltpu.VMEM_SHARED((ACC_ROWS, 16), jnp.float32),
        pltpu.VMEM((C,), jnp.int32),
        pltpu.VMEM((C,), jnp.int32),
        pltpu.VMEM((C, 16), jnp.float32),
    ],
)(_deg_body)


def _agg_body(y_hbm, src_hbm, dst_hbm, out_hbm, acc, src_buf, dst_buf,
              idx_buf, rows_buf, sem):
    c = lax.axis_index("c")
    s = lax.axis_index("s")
    base = c * HALF
    pltpu.sync_copy(y_hbm.at[pl.ds(base + s * RPS, RPS)],
                    acc.at[pl.ds(s * RPS, RPS)])
    plsc.subcore_barrier()

    def chunk(j, carry):
        off = s * EPT + j * C
        pltpu.sync_copy(src_hbm.at[pl.ds(off, C)], src_buf)
        pltpu.sync_copy(dst_hbm.at[pl.ds(off, C)], dst_buf)
        for k in range(C // 16):
            d = dst_buf[pl.ds(k * 16, 16)] - base
            ok = (d >= 0) & (d < HALF)
            idx_buf[pl.ds(k * 16, 16)] = jnp.where(ok, d, HALF)
        pltpu.async_copy(y_hbm.at[src_buf], rows_buf, sem).wait()
        pltpu.sync_copy(rows_buf, acc.at[idx_buf], add=True)
        return carry

    lax.fori_loop(0, EPT // C, chunk, 0)
    plsc.subcore_barrier()
    pltpu.sync_copy(acc.at[pl.ds(s * RPS, RPS)],
                    out_hbm.at[pl.ds(base + s * RPS, RPS)])


_agg_kernel = functools.partial(
    pl.kernel,
    out_type=jax.ShapeDtypeStruct((NP, H), jnp.float32),
    mesh=_mesh,
    scratch_types=[
        pltpu.VMEM_SHARED((ACC_ROWS, H), jnp.float32),
        pltpu.VMEM((C,), jnp.int32),
        pltpu.VMEM((C,), jnp.int32),
        pltpu.VMEM((C,), jnp.int32),
        pltpu.VMEM((C, H), jnp.float32),
        pltpu.SemaphoreType.DMA,
    ],
)(_agg_body)


def _pre_body(x_ref, w_ref, deg_ref, y_ref, dinv_ref):
    deg = deg_ref[...][:, 0:1] + 1.0
    dinv = lax.rsqrt(deg)
    h = jnp.dot(x_ref[...], w_ref[...], preferred_element_type=jnp.float32)
    y_ref[...] = h * dinv
    dinv_ref[...] = dinv


def _k_pre(x_p, w1_p, deg16):
    return pl.pallas_call(
        _pre_body,
        grid=(NP // BR,),
        in_specs=[
            pl.BlockSpec((BR, 8), lambda i: (i, 0)),
            pl.BlockSpec((8, H), lambda i: (0, 0)),
            pl.BlockSpec((BR, 16), lambda i: (i, 0)),
        ],
        out_specs=[
            pl.BlockSpec((BR, H), lambda i: (i, 0)),
            pl.BlockSpec((BR, 1), lambda i: (i, 0)),
        ],
        out_shape=[
            jax.ShapeDtypeStruct((NP, H), jnp.float32),
            jax.ShapeDtypeStruct((NP, 1), jnp.float32),
        ],
    )(x_p, w1_p, deg16)


def _layernorm_rows(t, g, be):
    m = jnp.mean(t, axis=-1, keepdims=True)
    v = jnp.mean((t - m) ** 2, axis=-1, keepdims=True)
    return (t - m) * lax.rsqrt(v + 1e-5) * g + be


def _mid_body(agg_ref, dinv_ref, b_ref, g_ref, be_ref, w_ref, y_ref):
    dinv = dinv_ref[...]
    t = agg_ref[...] * dinv + b_ref[...]
    t = _layernorm_rows(t, g_ref[...], be_ref[...])
    t = jnp.maximum(t, 0.0)
    y_ref[...] = jnp.dot(t, w_ref[...],
                         preferred_element_type=jnp.float32) * dinv


def _k_mid(agg, dinv, b, g, be, w_next):
    return pl.pallas_call(
        _mid_body,
        grid=(NP // BR,),
        in_specs=[
            pl.BlockSpec((BR, H), lambda i: (i, 0)),
            pl.BlockSpec((BR, 1), lambda i: (i, 0)),
            pl.BlockSpec((1, H), lambda i: (0, 0)),
            pl.BlockSpec((1, H), lambda i: (0, 0)),
            pl.BlockSpec((1, H), lambda i: (0, 0)),
            pl.BlockSpec((H, H), lambda i: (0, 0)),
        ],
        out_specs=pl.BlockSpec((BR, H), lambda i: (i, 0)),
        out_shape=jax.ShapeDtypeStruct((NP, H), jnp.float32),
    )(agg, dinv, b.reshape(1, H), g.reshape(1, H), be.reshape(1, H), w_next)


def _head_body(agg_ref, dinv_ref, b_ref, g_ref, be_ref, w4_ref, b4_ref,
               w5_ref, b5_ref, out_ref):
    t = agg_ref[...] * dinv_ref[...] + b_ref[...]
    t = _layernorm_rows(t, g_ref[...], be_ref[...])
    r = jnp.dot(t, w4_ref[...], preferred_element_type=jnp.float32)
    r = jnp.maximum(r + b4_ref[...], 0.0)
    o = jnp.dot(r, w5_ref[...], preferred_element_type=jnp.float32)
    out_ref[...] = jax.nn.sigmoid(o + b5_ref[...])


def _k_head(agg, dinv, b3, g3, be3, w4, b4, w5, b5):
    return pl.pallas_call(
        _head_body,
        grid=(NP // BR,),
        in_specs=[
            pl.BlockSpec((BR, H), lambda i: (i, 0)),
            pl.BlockSpec((BR, 1), lambda i: (i, 0)),
            pl.BlockSpec((1, H), lambda i: (0, 0)),
            pl.BlockSpec((1, H), lambda i: (0, 0)),
            pl.BlockSpec((1, H), lambda i: (0, 0)),
            pl.BlockSpec((H, H // 2), lambda i: (0, 0)),
            pl.BlockSpec((1, H // 2), lambda i: (0, 0)),
            pl.BlockSpec((H // 2, 1), lambda i: (0, 0)),
            pl.BlockSpec((1, 1), lambda i: (0, 0)),
        ],
        out_specs=pl.BlockSpec((BR, 1), lambda i: (i, 0)),
        out_shape=jax.ShapeDtypeStruct((NP, 1), jnp.float32),
    )(agg, dinv, b3.reshape(1, H), g3.reshape(1, H), be3.reshape(1, H),
      w4, b4.reshape(1, H // 2), w5, b5.reshape(1, 1))


def kernel(x, edge_index, W1, b1, W2, b2, W3, b3, g1, be1, g2, be2, g3, be3,
           W4, b4, W5, b5):
    src = edge_index[0]
    dst = edge_index[1]
    src_p = jnp.concatenate([src, jnp.zeros((EP - E,), jnp.int32)])
    dst_p = jnp.concatenate([dst, jnp.full((EP - E,), NP, jnp.int32)])
    x_p = jnp.pad(x, ((0, NP - N), (0, 8 - IN)))
    w1_p = jnp.pad(W1, ((0, 8 - IN), (0, 0)))
    zeros16 = jnp.zeros((ZR, 16), jnp.float32)
    ones16 = jnp.ones((C, 16), jnp.float32)

    deg16 = _deg_kernel(dst_p, zeros16, ones16)
    y1, dinv = _k_pre(x_p, w1_p, deg16)
    agg1 = _agg_kernel(y1, src_p, dst_p)
    y2 = _k_mid(agg1, dinv, b1, g1, be1, W2)
    agg2 = _agg_kernel(y2, src_p, dst_p)
    y3 = _k_mid(agg2, dinv, b2, g2, be2, W3)
    agg3 = _agg_kernel(y3, src_p, dst_p)
    out = _k_head(agg3, dinv, b3, g3, be3, W4, b4, W5, b5)
    return out[:N]


# speedup vs baseline: 2.0157x; 2.0157x over previous
import functools

import jax
import jax.numpy as jnp
from jax import lax
from jax.experimental import pallas as pl
from jax.experimental.pallas import tpu as pltpu
from jax.experimental.pallas import tpu_sc as plsc

N = 50000
E = 800000
H = 64
IN = 6
HP = 128

NC = 2
NS = 16

G = 4
GR = 12800
NP = G * GR
RPS = GR // NS
ACC_ROWS = GR + 16

C = 128
EPT = 51200
EP = NS * EPT

BR = 512

_mesh = plsc.VectorSubcoreMesh(
    core_axis_name="c", subcore_axis_name="s", num_cores=NC, num_subcores=NS
)


def _deg_body(dst_hbm, zeros_hbm, ones_hbm, deg_hbm, accd, dst_buf, idx_buf,
              ones_buf):
    c = lax.axis_index("c")
    s = lax.axis_index("s")
    trash = GR + s
    pltpu.sync_copy(ones_hbm, ones_buf)
    for q in range(2):
        gbase = (c * 2 + q) * GR
        pltpu.sync_copy(zeros_hbm, accd.at[pl.ds(s * RPS, RPS)])
        plsc.subcore_barrier()

        def chunk(j, carry):
            off = s * EPT + j * C
            pltpu.sync_copy(dst_hbm.at[pl.ds(off, C)], dst_buf)
            for k in range(C // 16):
                d = dst_buf[pl.ds(k * 16, 16)] - gbase
                ok = (d >= 0) & (d < GR)
                idx_buf[pl.ds(k * 16, 16)] = jnp.where(ok, d, trash)
            pltpu.sync_copy(ones_buf, accd.at[idx_buf], add=True)
            return carry

        lax.fori_loop(0, EPT // C, chunk, 0)
        plsc.subcore_barrier()
        pltpu.sync_copy(accd.at[pl.ds(s * RPS, RPS)],
                        deg_hbm.at[pl.ds(gbase + s * RPS, RPS)])
        plsc.subcore_barrier()


_deg_kernel = functools.partial(
    pl.kernel,
    out_type=jax.ShapeDtypeStruct((NP, HP), jnp.float32),
    mesh=_mesh,
    scratch_types=[
        pltpu.VMEM_SHARED((ACC_ROWS, HP), jnp.float32),
        pltpu.VMEM((C,), jnp.int32),
        pltpu.VMEM((C,), jnp.int32),
        pltpu.VMEM((C, HP), jnp.float32),
    ],
)(_deg_body)


def _agg_body(y_hbm, src_hbm, dst_hbm, out_hbm, acc, src_buf, dst_buf,
              idx_buf, rows_buf, sem):
    c = lax.axis_index("c")
    s = lax.axis_index("s")
    trash = GR + s
    for q in range(2):
        gbase = (c * 2 + q) * GR
        pltpu.sync_copy(y_hbm.at[pl.ds(gbase + s * RPS, RPS)],
                        acc.at[pl.ds(s * RPS, RPS)])
        plsc.subcore_barrier()

        def chunk(j, carry):
            off = s * EPT + j * C
            pltpu.sync_copy(src_hbm.at[pl.ds(off, C)], src_buf)
            pltpu.sync_copy(dst_hbm.at[pl.ds(off, C)], dst_buf)
            for k in range(C // 16):
                d = dst_buf[pl.ds(k * 16, 16)] - gbase
                ok = (d >= 0) & (d < GR)
                idx_buf[pl.ds(k * 16, 16)] = jnp.where(ok, d, trash)
            pltpu.async_copy(y_hbm.at[src_buf], rows_buf, sem).wait()
            pltpu.sync_copy(rows_buf, acc.at[idx_buf], add=True)
            return carry

        lax.fori_loop(0, EPT // C, chunk, 0)
        plsc.subcore_barrier()
        pltpu.sync_copy(acc.at[pl.ds(s * RPS, RPS)],
                        out_hbm.at[pl.ds(gbase + s * RPS, RPS)])
        plsc.subcore_barrier()


_agg_kernel = functools.partial(
    pl.kernel,
    out_type=jax.ShapeDtypeStruct((NP, HP), jnp.float32),
    mesh=_mesh,
    scratch_types=[
        pltpu.VMEM_SHARED((ACC_ROWS, HP), jnp.float32),
        pltpu.VMEM((C,), jnp.int32),
        pltpu.VMEM((C,), jnp.int32),
        pltpu.VMEM((C,), jnp.int32),
        pltpu.VMEM((C, HP), jnp.float32),
        pltpu.SemaphoreType.DMA,
    ],
)(_agg_body)


def _pre_body(x_ref, w_ref, deg_ref, y_ref, dinv_ref):
    deg = deg_ref[:, 0:1] + 1.0
    dinv = lax.rsqrt(deg)
    h = jnp.dot(x_ref[...], w_ref[...], preferred_element_type=jnp.float32)
    y_ref[:, :H] = h * dinv
    y_ref[:, H:] = jnp.zeros((BR, HP - H), jnp.float32)
    dinv_ref[...] = dinv


def _k_pre(x_p, w1_p, deg16):
    return pl.pallas_call(
        _pre_body,
        grid=(NP // BR,),
        in_specs=[
            pl.BlockSpec((BR, 8), lambda i: (i, 0)),
            pl.BlockSpec((8, H), lambda i: (0, 0)),
            pl.BlockSpec((BR, HP), lambda i: (i, 0)),
        ],
        out_specs=[
            pl.BlockSpec((BR, HP), lambda i: (i, 0)),
            pl.BlockSpec((BR, 1), lambda i: (i, 0)),
        ],
        out_shape=[
            jax.ShapeDtypeStruct((NP, HP), jnp.float32),
            jax.ShapeDtypeStruct((NP, 1), jnp.float32),
        ],
    )(x_p, w1_p, deg16)


def _layernorm_rows(t, g, be):
    m = jnp.mean(t, axis=-1, keepdims=True)
    v = jnp.mean((t - m) ** 2, axis=-1, keepdims=True)
    return (t - m) * lax.rsqrt(v + 1e-5) * g + be


def _mid_body(agg_ref, dinv_ref, b_ref, g_ref, be_ref, w_ref, y_ref):
    dinv = dinv_ref[...]
    t = agg_ref[:, :H] * dinv + b_ref[...]
    t = _layernorm_rows(t, g_ref[...], be_ref[...])
    t = jnp.maximum(t, 0.0)
    y_ref[:, :H] = jnp.dot(t, w_ref[...],
                           preferred_element_type=jnp.float32) * dinv
    y_ref[:, H:] = jnp.zeros((BR, HP - H), jnp.float32)


def _k_mid(agg, dinv, b, g, be, w_next):
    return pl.pallas_call(
        _mid_body,
        grid=(NP // BR,),
        in_specs=[
            pl.BlockSpec((BR, HP), lambda i: (i, 0)),
            pl.BlockSpec((BR, 1), lambda i: (i, 0)),
            pl.BlockSpec((1, H), lambda i: (0, 0)),
            pl.BlockSpec((1, H), lambda i: (0, 0)),
            pl.BlockSpec((1, H), lambda i: (0, 0)),
            pl.BlockSpec((H, H), lambda i: (0, 0)),
        ],
        out_specs=pl.BlockSpec((BR, HP), lambda i: (i, 0)),
        out_shape=jax.ShapeDtypeStruct((NP, HP), jnp.float32),
    )(agg, dinv, b.reshape(1, H), g.reshape(1, H), be.reshape(1, H), w_next)


def _head_body(agg_ref, dinv_ref, b_ref, g_ref, be_ref, w4_ref, b4_ref,
               w5_ref, b5_ref, out_ref):
    t = agg_ref[:, :H] * dinv_ref[...] + b_ref[...]
    t = _layernorm_rows(t, g_ref[...], be_ref[...])
    r = jnp.dot(t, w4_ref[...], preferred_element_type=jnp.float32)
    r = jnp.maximum(r + b4_ref[...], 0.0)
    o = jnp.dot(r, w5_ref[...], preferred_element_type=jnp.float32)
    out_ref[...] = jax.nn.sigmoid(o + b5_ref[...])


def _k_head(agg, dinv, b3, g3, be3, w4, b4, w5, b5):
    return pl.pallas_call(
        _head_body,
        grid=(NP // BR,),
        in_specs=[
            pl.BlockSpec((BR, HP), lambda i: (i, 0)),
            pl.BlockSpec((BR, 1), lambda i: (i, 0)),
            pl.BlockSpec((1, H), lambda i: (0, 0)),
            pl.BlockSpec((1, H), lambda i: (0, 0)),
            pl.BlockSpec((1, H), lambda i: (0, 0)),
            pl.BlockSpec((H, H // 2), lambda i: (0, 0)),
            pl.BlockSpec((1, H // 2), lambda i: (0, 0)),
            pl.BlockSpec((H // 2, 1), lambda i: (0, 0)),
            pl.BlockSpec((1, 1), lambda i: (0, 0)),
        ],
        out_specs=pl.BlockSpec((BR, 1), lambda i: (i, 0)),
        out_shape=jax.ShapeDtypeStruct((NP, 1), jnp.float32),
    )(agg, dinv, b3.reshape(1, H), g3.reshape(1, H), be3.reshape(1, H),
      w4, b4.reshape(1, H // 2), w5, b5.reshape(1, 1))


def kernel(x, edge_index, W1, b1, W2, b2, W3, b3, g1, be1, g2, be2, g3, be3,
           W4, b4, W5, b5):
    src = edge_index[0]
    dst = edge_index[1]
    src_p = jnp.concatenate([src, jnp.zeros((EP - E,), jnp.int32)])
    dst_p = jnp.concatenate([dst, jnp.full((EP - E,), NP, jnp.int32)])
    x_p = jnp.pad(x, ((0, NP - N), (0, 8 - IN)))
    w1_p = jnp.pad(W1, ((0, 8 - IN), (0, 0)))
    zeros128 = jnp.zeros((RPS, HP), jnp.float32)
    ones128 = jnp.ones((C, HP), jnp.float32)

    deg = _deg_kernel(dst_p, zeros128, ones128)
    y1, dinv = _k_pre(x_p, w1_p, deg)
    agg1 = _agg_kernel(y1, src_p, dst_p)
    y2 = _k_mid(agg1, dinv, b1, g1, be1, W2)
    agg2 = _agg_kernel(y2, src_p, dst_p)
    y3 = _k_mid(agg2, dinv, b2, g2, be2, W3)
    agg3 = _agg_kernel(y3, src_p, dst_p)
    out = _k_head(agg3, dinv, b3, g3, be3, W4, b4, W5, b5)
    return out[:N]


# speedup vs baseline: 4.1334x; 2.0507x over previous
import functools

import jax
import jax.numpy as jnp
from jax import lax
from jax.experimental import pallas as pl
from jax.experimental.pallas import tpu as pltpu
from jax.experimental.pallas import tpu_sc as plsc

N = 50000
E = 800000
H = 64
IN = 6
HP = 128

NC = 2
NS = 16
NW = NC * NS

G = 4
GR = 12800
NP = G * GR
RPS = GR // NS
ACC_ROWS = GR + 16

C = 128
CA = 64
EP = 802816
EPA = EP + C
ES = EP // NS
NCH = ES // CA
DS = EP // NW
DNCH = DS // C

NPH = NP + 512
BR = 512

_mesh = plsc.VectorSubcoreMesh(
    core_axis_name="c", subcore_axis_name="s", num_cores=NC, num_subcores=NS
)
_sc_params = pltpu.CompilerParams(needs_layout_passes=False)


def _deg_body(dst_hbm, zeros_hbm, parts_hbm, hist, dst_c):
    c = lax.axis_index("c")
    s = lax.axis_index("s")
    w = c * NS + s
    base = w * DS
    pltpu.sync_copy(zeros_hbm, hist)
    ones16 = jnp.ones((16,), jnp.float32)

    def chunk(j, carry):
        pltpu.sync_copy(dst_hbm.at[pl.ds(base + j * C, C)], dst_c)
        for k in range(C // 16):
            d16 = dst_c[pl.ds(k * 16, 16)]
            plsc.addupdate_scatter(hist, [d16], ones16)
        return carry

    lax.fori_loop(0, DNCH, chunk, 0)
    pltpu.sync_copy(hist, parts_hbm.at[w])


_deg_kernel = functools.partial(
    pl.kernel,
    out_type=jax.ShapeDtypeStruct((NW, NPH), jnp.float32),
    mesh=_mesh,
    compiler_params=_sc_params,
    scratch_types=[
        pltpu.VMEM((NPH,), jnp.float32),
        pltpu.VMEM((C,), jnp.int32),
    ],
)(_deg_body)


def _agg_body(y_hbm, src_hbm, dst_hbm, out_hbm, acc,
              src_a, src_b, dst_a, dst_b, rows_a, rows_b, sem_a, sem_b):
    c = lax.axis_index("c")
    s = lax.axis_index("s")
    base = s * ES
    trash16 = jnp.full((16,), GR, jnp.int32) + lax.iota(jnp.int32, 16)

    def load_remap(off, src_buf, dst_buf, gbase):
        pltpu.sync_copy(src_hbm.at[pl.ds(off, CA)], src_buf)
        pltpu.sync_copy(dst_hbm.at[pl.ds(off, CA)], dst_buf)
        for k in range(CA // 16):
            d16 = dst_buf[pl.ds(k * 16, 16)]
            lo = d16 - gbase
            m = (lo >= 0) & (lo < GR)
            dst_buf[pl.ds(k * 16, 16)] = jnp.where(m, lo, trash16)

    for q in range(2):
        g = c * 2 + q
        gbase = g * GR
        pltpu.sync_copy(y_hbm.at[pl.ds(gbase + s * RPS, RPS)],
                        acc.at[pl.ds(s * RPS, RPS)])
        plsc.subcore_barrier()

        load_remap(base, src_a, dst_a, gbase)
        pltpu.async_copy(y_hbm.at[src_a], rows_a, sem_a)

        def pair(i, carry):
            load_remap(base + (2 * i + 1) * CA, src_b, dst_b, gbase)
            pltpu.async_copy(y_hbm.at[src_b], rows_b, sem_b)
            pltpu.make_async_copy(y_hbm.at[src_a], rows_a, sem_a).wait()
            pltpu.sync_copy(rows_a, acc.at[dst_a], add=True)
            load_remap(base + (2 * i + 2) * CA, src_a, dst_a, gbase)
            pltpu.async_copy(y_hbm.at[src_a], rows_a, sem_a)
            pltpu.make_async_copy(y_hbm.at[src_b], rows_b, sem_b).wait()
            pltpu.sync_copy(rows_b, acc.at[dst_b], add=True)
            return carry

        lax.fori_loop(0, NCH // 2, pair, 0)
        pltpu.make_async_copy(y_hbm.at[src_a], rows_a, sem_a).wait()

        plsc.subcore_barrier()
        pltpu.sync_copy(acc.at[pl.ds(s * RPS, RPS)],
                        out_hbm.at[pl.ds(gbase + s * RPS, RPS)])
        plsc.subcore_barrier()


_agg_kernel = functools.partial(
    pl.kernel,
    out_type=jax.ShapeDtypeStruct((NP, HP), jnp.float32),
    mesh=_mesh,
    compiler_params=_sc_params,
    scratch_types=[
        pltpu.VMEM_SHARED((ACC_ROWS, HP), jnp.float32),
        pltpu.VMEM((CA,), jnp.int32),
        pltpu.VMEM((CA,), jnp.int32),
        pltpu.VMEM((CA,), jnp.int32),
        pltpu.VMEM((CA,), jnp.int32),
        pltpu.VMEM((CA, HP), jnp.float32),
        pltpu.VMEM((CA, HP), jnp.float32),
        pltpu.SemaphoreType.DMA,
        pltpu.SemaphoreType.DMA,
    ],
)(_agg_body)


def _degsum_body(parts_ref, out_ref):
    out_ref[...] = jnp.sum(parts_ref[...], axis=0, keepdims=True)


def _k_degsum(parts):
    return pl.pallas_call(
        _degsum_body,
        grid=(NPH // BR,),
        in_specs=[pl.BlockSpec((NW, BR), lambda i: (0, i))],
        out_specs=pl.BlockSpec((1, BR), lambda i: (0, i)),
        out_shape=jax.ShapeDtypeStruct((1, NPH), jnp.float32),
    )(parts)


def _pre_body(x_ref, w_ref, deg_ref, y_ref, dinv_ref):
    deg = deg_ref[...] + 1.0
    dinv = lax.rsqrt(deg)
    h = jnp.dot(x_ref[...], w_ref[...], preferred_element_type=jnp.float32)
    y_ref[:, :H] = h * dinv
    y_ref[:, H:] = jnp.zeros((BR, HP - H), jnp.float32)
    dinv_ref[...] = dinv


def _k_pre(x_p, w1_p, deg_col):
    return pl.pallas_call(
        _pre_body,
        grid=(NP // BR,),
        in_specs=[
            pl.BlockSpec((BR, 8), lambda i: (i, 0)),
            pl.BlockSpec((8, H), lambda i: (0, 0)),
            pl.BlockSpec((BR, 1), lambda i: (i, 0)),
        ],
        out_specs=[
            pl.BlockSpec((BR, HP), lambda i: (i, 0)),
            pl.BlockSpec((BR, 1), lambda i: (i, 0)),
        ],
        out_shape=[
            jax.ShapeDtypeStruct((NP, HP), jnp.float32),
            jax.ShapeDtypeStruct((NP, 1), jnp.float32),
        ],
    )(x_p, w1_p, deg_col)


def _layernorm_rows(t, g, be):
    m = jnp.mean(t, axis=-1, keepdims=True)
    v = jnp.mean((t - m) ** 2, axis=-1, keepdims=True)
    return (t - m) * lax.rsqrt(v + 1e-5) * g + be


def _mid_body(agg_ref, dinv_ref, b_ref, g_ref, be_ref, w_ref, y_ref):
    dinv = dinv_ref[...]
    t = agg_ref[:, :H] * dinv + b_ref[...]
    t = _layernorm_rows(t, g_ref[...], be_ref[...])
    t = jnp.maximum(t, 0.0)
    y_ref[:, :H] = jnp.dot(t, w_ref[...],
                           preferred_element_type=jnp.float32) * dinv
    y_ref[:, H:] = jnp.zeros((BR, HP - H), jnp.float32)


def _k_mid(agg, dinv, b, g, be, w_next):
    return pl.pallas_call(
        _mid_body,
        grid=(NP // BR,),
        in_specs=[
            pl.BlockSpec((BR, HP), lambda i: (i, 0)),
            pl.BlockSpec((BR, 1), lambda i: (i, 0)),
            pl.BlockSpec((1, H), lambda i: (0, 0)),
            pl.BlockSpec((1, H), lambda i: (0, 0)),
            pl.BlockSpec((1, H), lambda i: (0, 0)),
            pl.BlockSpec((H, H), lambda i: (0, 0)),
        ],
        out_specs=pl.BlockSpec((BR, HP), lambda i: (i, 0)),
        out_shape=jax.ShapeDtypeStruct((NP, HP), jnp.float32),
    )(agg, dinv, b.reshape(1, H), g.reshape(1, H), be.reshape(1, H), w_next)


def _head_body(agg_ref, dinv_ref, b_ref, g_ref, be_ref, w4_ref, b4_ref,
               w5_ref, b5_ref, out_ref):
    t = agg_ref[:, :H] * dinv_ref[...] + b_ref[...]
    t = _layernorm_rows(t, g_ref[...], be_ref[...])
    r = jnp.dot(t, w4_ref[...], preferred_element_type=jnp.float32)
    r = jnp.maximum(r + b4_ref[...], 0.0)
    o = jnp.dot(r, w5_ref[...], preferred_element_type=jnp.float32)
    out_ref[...] = jax.nn.sigmoid(o + b5_ref[...])


def _k_head(agg, dinv, b3, g3, be3, w4, b4, w5, b5):
    return pl.pallas_call(
        _head_body,
        grid=(NP // BR,),
        in_specs=[
            pl.BlockSpec((BR, HP), lambda i: (i, 0)),
            pl.BlockSpec((BR, 1), lambda i: (i, 0)),
            pl.BlockSpec((1, H), lambda i: (0, 0)),
            pl.BlockSpec((1, H), lambda i: (0, 0)),
            pl.BlockSpec((1, H), lambda i: (0, 0)),
            pl.BlockSpec((H, H // 2), lambda i: (0, 0)),
            pl.BlockSpec((1, H // 2), lambda i: (0, 0)),
            pl.BlockSpec((H // 2, 1), lambda i: (0, 0)),
            pl.BlockSpec((1, 1), lambda i: (0, 0)),
        ],
        out_specs=pl.BlockSpec((BR, 1), lambda i: (i, 0)),
        out_shape=jax.ShapeDtypeStruct((NP, 1), jnp.float32),
    )(agg, dinv, b3.reshape(1, H), g3.reshape(1, H), be3.reshape(1, H),
      w4, b4.reshape(1, H // 2), w5, b5.reshape(1, 1))


def kernel(x, edge_index, W1, b1, W2, b2, W3, b3, g1, be1, g2, be2, g3, be3,
           W4, b4, W5, b5):
    src = edge_index[0]
    dst = edge_index[1]
    src_p = jnp.concatenate([src, jnp.zeros((EPA - E,), jnp.int32)])
    dst_p = jnp.concatenate([dst, jnp.full((EPA - E,), NP, jnp.int32)])
    x_p = jnp.pad(x, ((0, NP - N), (0, 8 - IN)))
    w1_p = jnp.pad(W1, ((0, 8 - IN), (0, 0)))
    zeros_nph = jnp.zeros((NPH,), jnp.float32)

    parts = _deg_kernel(dst_p, zeros_nph)
    deg_col = _k_degsum(parts)[0, :NP].reshape(NP, 1)
    y1, dinv = _k_pre(x_p, w1_p, deg_col)
    agg1 = _agg_kernel(y1, src_p, dst_p)
    y2 = _k_mid(agg1, dinv, b1, g1, be1, W2)
    agg2 = _agg_kernel(y2, src_p, dst_p)
    y3 = _k_mid(agg2, dinv, b2, g2, be2, W3)
    agg3 = _agg_kernel(y3, src_p, dst_p)
    out = _k_head(agg3, dinv, b3, g3, be3, W4, b4, W5, b5)
    return out[:N]
